# initial kernel scaffold (unmeasured)
import jax
import jax.numpy as jnp
from jax import lax
from jax.experimental import pallas as pl
from jax.experimental.pallas import tpu as pltpu

N_DEV = 4


def kernel(x, w_mat, scale_x, scale_w):
    m_per, k = x.shape
    _, n_per = w_mat.shape
    half = m_per // 2
    m_full = N_DEV * m_per

    def body(x_ref, w_ref, sx_ref, sw_ref, out_ref, comm_ref, send_sems, recv_sems):
        my = lax.axis_index("i")
        left = lax.rem(my - 1 + N_DEV, N_DEV)
        right = lax.rem(my + 1, N_DEV)

        barrier_sem = pltpu.get_barrier_semaphore()
        for nbr in (left, right):
            pl.semaphore_signal(
                barrier_sem, inc=1,
                device_id=(nbr,), device_id_type=pl.DeviceIdType.MESH,
            )
        pl.semaphore_wait(barrier_sem, 2)

        scale = sx_ref[0] * sw_ref[0]

        def gemm(origin, data):
            acc = lax.dot_general(
                data, w_ref[...],
                (((1,), (0,)), ((), ())),
                preferred_element_type=jnp.float32,
            )
            out_ref[pl.ds(origin * m_per, m_per), :] = jnp.maximum(acc * scale, 0.0)

        send_r = pltpu.make_async_remote_copy(
            src_ref=x_ref, dst_ref=comm_ref.at[0],
            send_sem=send_sems.at[0], recv_sem=recv_sems.at[0],
            device_id=(right,), device_id_type=pl.DeviceIdType.MESH,
        )
        send_l = pltpu.make_async_remote_copy(
            src_ref=x_ref, dst_ref=comm_ref.at[1],
            send_sem=send_sems.at[1], recv_sem=recv_sems.at[1],
            device_id=(left,), device_id_type=pl.DeviceIdType.MESH,
        )
        send_r.start()
        send_l.start()

        gemm(my, x_ref[...])

        send_r.wait()
        send_l.wait()

        fwd_r = pltpu.make_async_remote_copy(
            src_ref=comm_ref.at[0, :half], dst_ref=comm_ref.at[2, :half],
            send_sem=send_sems.at[2], recv_sem=recv_sems.at[2],
            device_id=(right,), device_id_type=pl.DeviceIdType.MESH,
        )
        fwd_l = pltpu.make_async_remote_copy(
            src_ref=comm_ref.at[1, half:], dst_ref=comm_ref.at[2, half:],
            send_sem=send_sems.at[3], recv_sem=recv_sems.at[3],
            device_id=(left,), device_id_type=pl.DeviceIdType.MESH,
        )
        fwd_r.start()
        fwd_l.start()

        gemm(left, comm_ref[0])
        gemm(right, comm_ref[1])

        fwd_r.wait()
        fwd_l.wait()

        opp = lax.rem(my + 2, N_DEV)
        gemm(opp, comm_ref[2])

    return pl.pallas_call(
        body,
        out_shape=jax.ShapeDtypeStruct((m_full, n_per), jnp.float32),
        in_specs=[
            pl.BlockSpec(memory_space=pltpu.VMEM),
            pl.BlockSpec(memory_space=pltpu.VMEM),
            pl.BlockSpec(memory_space=pltpu.SMEM),
            pl.BlockSpec(memory_space=pltpu.SMEM),
        ],
        out_specs=pl.BlockSpec(memory_space=pltpu.VMEM),
        scratch_shapes=[
            pltpu.VMEM((3, m_per, k), x.dtype),
            pltpu.SemaphoreType.DMA((4,)),
            pltpu.SemaphoreType.DMA((4,)),
        ],
        compiler_params=pltpu.CompilerParams(collective_id=0),
    )(x, w_mat, scale_x, scale_w)


# baseline (device time: 144479 ns/iter reference)
import jax
import jax.numpy as jnp
from jax import lax
from jax.experimental import pallas as pl
from jax.experimental.pallas import tpu as pltpu

N_DEV = 4


def kernel(x, w_mat, scale_x, scale_w):
    x = x.astype(jnp.float8_e5m2)
    w_mat = w_mat.astype(jnp.float8_e5m2)
    m_per, k = x.shape
    _, n_per = w_mat.shape
    half = m_per // 2
    m_full = N_DEV * m_per

    def body(x_ref, w_ref, sx_ref, sw_ref, out_ref, comm_ref, send_sems, recv_sems):
        my = lax.axis_index("i")
        left = lax.rem(my - 1 + N_DEV, N_DEV)
        right = lax.rem(my + 1, N_DEV)

        barrier_sem = pltpu.get_barrier_semaphore()
        for nbr in (left, right):
            pl.semaphore_signal(
                barrier_sem, inc=1,
                device_id=(nbr,), device_id_type=pl.DeviceIdType.MESH,
            )
        pl.semaphore_wait(barrier_sem, 2)

        scale = sx_ref[0] * sw_ref[0]

        n_tile = 512

        def gemm(origin, data):
            for j in range(0, n_per, n_tile):
                acc = lax.dot_general(
                    data, w_ref[:, j:j + n_tile],
                    (((1,), (0,)), ((), ())),
                    preferred_element_type=jnp.float32,
                )
                out_ref[pl.ds(origin * m_per, m_per), j:j + n_tile] = (
                    jnp.maximum(acc * scale, 0.0)
                )

        send_r = pltpu.make_async_remote_copy(
            src_ref=x_ref, dst_ref=comm_ref.at[0],
            send_sem=send_sems.at[0], recv_sem=recv_sems.at[0],
            device_id=(right,), device_id_type=pl.DeviceIdType.MESH,
        )
        send_l = pltpu.make_async_remote_copy(
            src_ref=x_ref, dst_ref=comm_ref.at[1],
            send_sem=send_sems.at[1], recv_sem=recv_sems.at[1],
            device_id=(left,), device_id_type=pl.DeviceIdType.MESH,
        )
        send_r.start()
        send_l.start()

        gemm(my, x_ref[...])

        send_r.wait()
        send_l.wait()

        fwd_r = pltpu.make_async_remote_copy(
            src_ref=comm_ref.at[0, :half], dst_ref=comm_ref.at[2, :half],
            send_sem=send_sems.at[2], recv_sem=recv_sems.at[2],
            device_id=(right,), device_id_type=pl.DeviceIdType.MESH,
        )
        fwd_l = pltpu.make_async_remote_copy(
            src_ref=comm_ref.at[1, half:], dst_ref=comm_ref.at[2, half:],
            send_sem=send_sems.at[3], recv_sem=recv_sems.at[3],
            device_id=(left,), device_id_type=pl.DeviceIdType.MESH,
        )
        fwd_r.start()
        fwd_l.start()

        gemm(left, comm_ref[0])
        gemm(right, comm_ref[1])

        fwd_r.wait()
        fwd_l.wait()

        opp = lax.rem(my + 2, N_DEV)
        gemm(opp, comm_ref[2])

    return pl.pallas_call(
        body,
        out_shape=jax.ShapeDtypeStruct((m_full, n_per), jnp.float32),
        in_specs=[
            pl.BlockSpec(memory_space=pltpu.VMEM),
            pl.BlockSpec(memory_space=pltpu.VMEM),
            pl.BlockSpec(memory_space=pltpu.SMEM),
            pl.BlockSpec(memory_space=pltpu.SMEM),
        ],
        out_specs=pl.BlockSpec(memory_space=pltpu.VMEM),
        scratch_shapes=[
            pltpu.VMEM((3, m_per, k), x.dtype),
            pltpu.SemaphoreType.DMA((4,)),
            pltpu.SemaphoreType.DMA((4,)),
        ],
        compiler_params=pltpu.CompilerParams(
            collective_id=0,
            vmem_limit_bytes=63 * 1024 * 1024,
        ),
    )(x, w_mat, scale_x, scale_w)


# device time: 115378 ns/iter; 1.2522x vs baseline; 1.2522x over previous
import jax
import jax.numpy as jnp
from jax import lax
from jax.experimental import pallas as pl
from jax.experimental.pallas import tpu as pltpu

N_DEV = 4


def kernel(x, w_mat, scale_x, scale_w):
    m_per, k = x.shape
    _, n_per = w_mat.shape
    half = m_per // 2
    m_full = N_DEV * m_per
    n_tile = 512

    def body(x_ref, w_ref, sx_ref, sw_ref, out_ref,
             x8_ref, comm_ref, w8_ref, wstage_ref, ostage_ref,
             send_sems, recv_sems, wsems, osems):
        my = lax.axis_index("i")
        left = lax.rem(my - 1 + N_DEV, N_DEV)
        right = lax.rem(my + 1, N_DEV)

        barrier_sem = pltpu.get_barrier_semaphore()
        for nbr in (left, right):
            pl.semaphore_signal(
                barrier_sem, inc=1,
                device_id=(nbr,), device_id_type=pl.DeviceIdType.MESH,
            )
        pl.semaphore_wait(barrier_sem, 2)

        w_dma_tile = n_tile // 2

        def w_dma(j, slot):
            return pltpu.make_async_copy(
                w_ref.at[:, j * w_dma_tile:(j + 1) * w_dma_tile],
                wstage_ref.at[slot],
                wsems.at[slot],
            )

        w_dma(0, 0).start()

        x8_ref[...] = x_ref[...].astype(jnp.float8_e5m2)

        send_r = pltpu.make_async_remote_copy(
            src_ref=x8_ref, dst_ref=comm_ref.at[0],
            send_sem=send_sems.at[0], recv_sem=recv_sems.at[0],
            device_id=(right,), device_id_type=pl.DeviceIdType.MESH,
        )
        send_l = pltpu.make_async_remote_copy(
            src_ref=x8_ref, dst_ref=comm_ref.at[1],
            send_sem=send_sems.at[1], recv_sem=recv_sems.at[1],
            device_id=(left,), device_id_type=pl.DeviceIdType.MESH,
        )
        send_r.start()
        send_l.start()

        scale = sx_ref[0] * sw_ref[0]

        ocopy_inflight = [None, None]
        ocopy_count = [0, 0]

        def gemm_tile(origin, data, j):
            acc = lax.dot_general(
                data, w8_ref[:, j * n_tile:(j + 1) * n_tile],
                (((1,), (0,)), ((), ())),
                preferred_element_type=jnp.float32,
            )
            slot = ocopy_count[0] % 2
            ocopy_count[0] += 1
            if ocopy_inflight[slot] is not None:
                ocopy_inflight[slot].wait()
            ostage_ref[slot] = jnp.maximum(acc * scale, 0.0)
            cp = pltpu.make_async_copy(
                ostage_ref.at[slot],
                out_ref.at[pl.ds(origin * m_per, m_per),
                           j * n_tile:(j + 1) * n_tile],
                osems.at[slot],
            )
            cp.start()
            ocopy_inflight[slot] = cp

        def gemm(origin, data):
            for j in range(n_per // n_tile):
                gemm_tile(origin, data, j)

        n_w = n_per // w_dma_tile
        for j in range(n_per // n_tile):
            for h in range(2):
                jw = 2 * j + h
                if jw + 1 < n_w:
                    w_dma(jw + 1, (jw + 1) % 2).start()
                w_dma(jw, jw % 2).wait()
                w8_ref[:, jw * w_dma_tile:(jw + 1) * w_dma_tile] = (
                    wstage_ref[jw % 2].astype(jnp.float8_e5m2)
                )
            gemm_tile(my, x8_ref[...], j)

        send_r.wait()
        send_l.wait()

        fwd_r = pltpu.make_async_remote_copy(
            src_ref=comm_ref.at[0, :half], dst_ref=comm_ref.at[2, :half],
            send_sem=send_sems.at[2], recv_sem=recv_sems.at[2],
            device_id=(right,), device_id_type=pl.DeviceIdType.MESH,
        )
        fwd_l = pltpu.make_async_remote_copy(
            src_ref=comm_ref.at[1, half:], dst_ref=comm_ref.at[2, half:],
            send_sem=send_sems.at[3], recv_sem=recv_sems.at[3],
            device_id=(left,), device_id_type=pl.DeviceIdType.MESH,
        )
        fwd_r.start()
        fwd_l.start()

        gemm(left, comm_ref[0])
        gemm(right, comm_ref[1])

        fwd_r.wait()
        fwd_l.wait()

        opp = lax.rem(my + 2, N_DEV)
        gemm(opp, comm_ref[2])

        for cp in ocopy_inflight:
            if cp is not None:
                cp.wait()

    return pl.pallas_call(
        body,
        out_shape=jax.ShapeDtypeStruct((m_full, n_per), jnp.float32),
        in_specs=[
            pl.BlockSpec(memory_space=pltpu.VMEM),
            pl.BlockSpec(memory_space=pltpu.MemorySpace.HBM),
            pl.BlockSpec(memory_space=pltpu.SMEM),
            pl.BlockSpec(memory_space=pltpu.SMEM),
        ],
        out_specs=pl.BlockSpec(memory_space=pltpu.MemorySpace.HBM),
        scratch_shapes=[
            pltpu.VMEM((m_per, k), jnp.float8_e5m2),
            pltpu.VMEM((3, m_per, k), jnp.float8_e5m2),
            pltpu.VMEM((k, n_per), jnp.float8_e5m2),
            pltpu.VMEM((2, k, n_tile // 2), jnp.float32),
            pltpu.VMEM((2, m_per, n_tile), jnp.float32),
            pltpu.SemaphoreType.DMA((4,)),
            pltpu.SemaphoreType.DMA((4,)),
            pltpu.SemaphoreType.DMA((2,)),
            pltpu.SemaphoreType.DMA((2,)),
        ],
        compiler_params=pltpu.CompilerParams(
            collective_id=0,
            vmem_limit_bytes=63 * 1024 * 1024,
        ),
    )(x, w_mat, scale_x, scale_w)
